# baseline (speedup 1.0000x reference)
import jax
import jax.numpy as jnp
from jax.experimental import pallas as pl
from jax.experimental.pallas import tpu as pltpu

CLS = 1000
ROWS = 4096 * 26
R = 1024
NB = ROWS // R


def _onehot_body(x_ref, o_ref):
    idx = x_ref[0, 0, :]
    iota = jax.lax.broadcasted_iota(jnp.int32, (R, CLS), 1)
    o_ref[...] = (idx[:, None] == iota).astype(jnp.float32)


def kernel(x):
    xf = x.reshape(NB, 1, R)
    out = pl.pallas_call(
        _onehot_body,
        grid=(NB,),
        in_specs=[pl.BlockSpec((1, 1, R), lambda i: (i, 0, 0))],
        out_specs=pl.BlockSpec((R, CLS), lambda i: (i, 0)),
        out_shape=jax.ShapeDtypeStruct((ROWS, CLS), jnp.float32),
        compiler_params=pltpu.CompilerParams(
            dimension_semantics=("arbitrary",),
        ),
    )(xf)
    return out.reshape(x.shape + (CLS,))


# speedup vs baseline: 1.3854x; 1.3854x over previous
import jax
import jax.numpy as jnp
from jax.experimental import pallas as pl
from jax.experimental.pallas import tpu as pltpu

CLS = 1000
N0 = 4096
N1 = 26
BV = 32
NB = N0 // BV


def _onehot_body(x_ref, o_ref):
    idx = x_ref[...]
    iota = jax.lax.broadcasted_iota(jnp.int32, (BV, N1, CLS), 2)
    o_ref[...] = (idx[:, :, None] == iota).astype(jnp.float32)


def kernel(x):
    return pl.pallas_call(
        _onehot_body,
        grid=(NB,),
        in_specs=[pl.BlockSpec((BV, N1), lambda i: (i, 0))],
        out_specs=pl.BlockSpec((BV, N1, CLS), lambda i: (i, 0, 0)),
        out_shape=jax.ShapeDtypeStruct((N0, N1, CLS), jnp.float32),
        compiler_params=pltpu.CompilerParams(
            dimension_semantics=("arbitrary",),
        ),
    )(x)


# speedup vs baseline: 6.3177x; 4.5602x over previous
import jax
import jax.numpy as jnp
from jax.experimental import pallas as pl
from jax.experimental.pallas import tpu as pltpu

CLS = 1000
N0 = 4096
N1 = 26
BC = 200
NCB = CLS // BC


def _onehot_body(x_ref, o_ref):
    jc = pl.program_id(1)
    xrow = x_ref[0, 0, :]
    ci = jax.lax.broadcasted_iota(jnp.int32, (BC, N0), 0) + jc * BC
    o_ref[0] = (ci == xrow[None, :]).astype(jnp.float32)


def kernel(x):
    xt = x.T.reshape(N1, 1, N0)
    out_t = pl.pallas_call(
        _onehot_body,
        grid=(N1, NCB),
        in_specs=[pl.BlockSpec((1, 1, N0), lambda i, j: (i, 0, 0))],
        out_specs=pl.BlockSpec((1, BC, N0), lambda i, j: (i, j, 0)),
        out_shape=jax.ShapeDtypeStruct((N1, CLS, N0), jnp.float32),
        compiler_params=pltpu.CompilerParams(
            dimension_semantics=("arbitrary", "arbitrary"),
        ),
    )(xt)
    return jnp.transpose(out_t, (2, 0, 1))


# speedup vs baseline: 6.6441x; 1.0517x over previous
import jax
import jax.numpy as jnp
from jax.experimental import pallas as pl
from jax.experimental.pallas import tpu as pltpu

CLS = 1000
N0 = 4096
N1 = 26
BC = 200
NCB = CLS // BC


def _onehot_body(x_ref, o_ref):
    i1 = pl.program_id(0)
    jc = pl.program_id(1)
    xrow = x_ref[pl.ds(i1, 1), :]
    ci = jax.lax.broadcasted_iota(jnp.int32, (BC, N0), 0) + jc * BC
    o_ref[0] = (ci == xrow).astype(jnp.float32)


def kernel(x):
    xt = x.T
    out_t = pl.pallas_call(
        _onehot_body,
        grid=(N1, NCB),
        in_specs=[pl.BlockSpec((N1, N0), lambda i, j: (0, 0))],
        out_specs=pl.BlockSpec((1, BC, N0), lambda i, j: (i, j, 0)),
        out_shape=jax.ShapeDtypeStruct((N1, CLS, N0), jnp.float32),
        compiler_params=pltpu.CompilerParams(
            dimension_semantics=("arbitrary", "arbitrary"),
        ),
    )(xt)
    return jnp.transpose(out_t, (2, 0, 1))
